# baseline (speedup 1.0000x reference)
import functools

import jax
import jax.numpy as jnp
import numpy as np
from jax import lax
from jax.experimental import pallas as pl
from jax.experimental.pallas import tpu as pltpu
from jax.experimental.pallas import tpu_sc as plsc

_MIN_RADIUS = 0.05
_MAX_RADIUS = 0.3
_K = 16
_BM = 64

_DENOM = np.float32(4.0 / 3.0 * np.pi * _MIN_RADIUS ** 3 + 1e-08)


def _dist_block(pts_ref, ctr_ref):
    p = pts_ref[0]
    c = ctr_ref[0]
    dx = c[:, 0:1] - p[0:1, :]
    dy = c[:, 1:2] - p[1:2, :]
    dz = c[:, 2:3] - p[2:3, :]
    return jnp.sqrt(dx * dx + dy * dy + dz * dz)


def _count_body(pts_ref, ctr_ref, cnt_ref):
    dist = _dist_block(pts_ref, ctr_ref)
    mask = (dist < _MIN_RADIUS).astype(jnp.float32)
    cnt_ref[0, 0] = jnp.sum(mask, axis=1, keepdims=True)


_SUP = np.float32(3e38)
_BIGI = np.float32(1e9)
_R = 3


def _select_body(pts_ref, ctr_ref, cnt_blk_ref, cnt_full_ref, out_ref, v_ref, v2_ref):
    n = pts_ref.shape[2]
    nc = n // 128
    dist = _dist_block(pts_ref, ctr_ref)
    density_full = cnt_full_ref[...] / _DENOM
    density_max = jnp.max(density_full) + np.float32(1e-8)
    density = cnt_blk_ref[0, 0] / _DENOM
    radii = _MIN_RADIUS + (_MAX_RADIUS - _MIN_RADIUS) * (1.0 - density / density_max)
    v_ref[...] = jnp.where(dist < radii, dist, jnp.float32(1e10))
    b_off = pl.program_id(0) * n

    iota3 = (
        lax.broadcasted_iota(jnp.int32, (_BM, nc, 128), 1) * 128
        + lax.broadcasted_iota(jnp.int32, (_BM, nc, 128), 2)
    ).astype(jnp.float32)
    v2_ref[...] = v_ref[...]
    lane_v = []
    lane_i = []
    for _ in range(_R):
        v3 = v2_ref[...].reshape(_BM, nc, 128)
        m = jnp.min(v3, axis=1, keepdims=True)
        cand = jnp.where(v3 == m, iota3, _BIGI)
        ji = jnp.min(cand, axis=1, keepdims=True)
        lane_v.append(m.reshape(_BM, 128))
        lane_i.append(ji.reshape(_BM, 128))
        v2_ref[...] = jnp.where(iota3 == ji, _SUP, v3).reshape(_BM, n)

    w, wi = lane_v[0], lane_i[0]
    lvl = jnp.zeros((_BM, 128), jnp.float32)
    t = None
    for k in range(_K):
        m = jnp.min(w, axis=1, keepdims=True)
        cand = jnp.where(w == m, wi, _BIGI)
        ji = jnp.min(cand, axis=1, keepdims=True)
        out_ref[0, 0, :, k : k + 1] = ji.astype(jnp.int32) + b_off
        t = m
        wn = jnp.where(lvl == 0.0, lane_v[1], jnp.where(lvl == 1.0, lane_v[2], _SUP))
        win = jnp.where(lvl == 0.0, lane_i[1], jnp.where(lvl == 1.0, lane_i[2], _BIGI))
        hit = (w == m) & (wi == ji)
        w = jnp.where(hit, wn, w)
        wi = jnp.where(hit, win, wi)
        lvl = lvl + hit.astype(jnp.float32)

    rem3 = (v2_ref[...].reshape(_BM, nc, 128) <= t[:, :, None]).astype(jnp.float32)
    rem = jnp.sum(rem3, axis=1)
    suspect = (lvl >= float(_R)) & (rem >= 1.0)
    any_suspect = jnp.max(suspect.astype(jnp.int32))

    @pl.when(any_suspect == 1)
    def _slow():
        iota = lax.broadcasted_iota(jnp.int32, (_BM, n), 1).astype(jnp.float32)
        for k in range(_K):
            v = v_ref[...]
            m = jnp.min(v, axis=1, keepdims=True)
            cand = jnp.where(v == m, iota, _BIGI)
            ji = jnp.min(cand, axis=1, keepdims=True)
            out_ref[0, 0, :, k : k + 1] = ji.astype(jnp.int32) + b_off
            v_ref[...] = jnp.where(iota == ji, _SUP, v)


def _sc_gather_body(per_w, ch, feat_ref, idx_ref, out_ref, idx_v, rows_v, sem):
    nc = lax.axis_size("c")
    wid = lax.axis_index("s") * nc + lax.axis_index("c")
    base = wid * per_w
    for i in range(per_w // ch):
        off = base + i * ch
        pltpu.sync_copy(idx_ref.at[pl.ds(off, ch)], idx_v)
        pltpu.async_copy(feat_ref.at[idx_v], rows_v, sem).wait()
        pltpu.sync_copy(rows_v, out_ref.at[pl.ds(off, ch)])


def kernel(points, features, center_indices):
    B, N, _ = points.shape
    M = center_indices.shape[1]
    C = features.shape[2]
    MB = M // _BM

    pointsT = points.transpose(0, 2, 1)
    centers = jnp.take_along_axis(
        points, jnp.broadcast_to(center_indices[:, :, None], (B, M, 3)), axis=1
    )

    counts = pl.pallas_call(
        _count_body,
        grid=(B, MB),
        in_specs=[
            pl.BlockSpec((1, 3, N), lambda b, mb: (b, 0, 0)),
            pl.BlockSpec((1, _BM, 3), lambda b, mb: (b, mb, 0)),
        ],
        out_specs=pl.BlockSpec((1, 1, _BM, 1), lambda b, mb: (b, mb, 0, 0)),
        out_shape=jax.ShapeDtypeStruct((B, MB, _BM, 1), jnp.float32),
    )(pointsT, centers)

    knn_idx = pl.pallas_call(
        _select_body,
        grid=(B, MB),
        in_specs=[
            pl.BlockSpec((1, 3, N), lambda b, mb: (b, 0, 0)),
            pl.BlockSpec((1, _BM, 3), lambda b, mb: (b, mb, 0)),
            pl.BlockSpec((1, 1, _BM, 1), lambda b, mb: (b, mb, 0, 0)),
            pl.BlockSpec((B, MB, _BM, 1), lambda b, mb: (0, 0, 0, 0)),
        ],
        out_specs=pl.BlockSpec((1, 1, _BM, _K), lambda b, mb: (b, mb, 0, 0)),
        out_shape=jax.ShapeDtypeStruct((B, MB, _BM, _K), jnp.int32),
        scratch_shapes=[
            pltpu.VMEM((_BM, N), jnp.float32),
            pltpu.VMEM((_BM, N), jnp.float32),
        ],
    )(pointsT, centers, counts, counts)

    tot = B * M * _K
    idx_flat = knn_idx.reshape(tot)
    feat_flat = features.reshape(B * N, C)

    info = plsc.get_sparse_core_info()
    nw = info.num_cores * info.num_subcores
    per_w = tot // nw
    ch = 128
    gather = pl.kernel(
        functools.partial(_sc_gather_body, per_w, ch),
        out_type=jax.ShapeDtypeStruct((tot, C), jnp.float32),
        mesh=plsc.VectorSubcoreMesh(core_axis_name="c", subcore_axis_name="s"),
        compiler_params=pltpu.CompilerParams(use_tc_tiling_on_sc=False),
        scratch_types=[
            pltpu.VMEM((ch,), jnp.int32),
            pltpu.VMEM((ch, C), jnp.float32),
            pltpu.SemaphoreType.DMA,
        ],
    )
    out_flat = gather(feat_flat, idx_flat)
    return out_flat.reshape(B, M, _K, C)


# speedup vs baseline: 1.5341x; 1.5341x over previous
import functools

import jax
import jax.numpy as jnp
import numpy as np
from jax import lax
from jax.experimental import pallas as pl
from jax.experimental.pallas import tpu as pltpu
from jax.experimental.pallas import tpu_sc as plsc

_MIN_RADIUS = 0.05
_MAX_RADIUS = 0.3
_K = 16
_BM = 64

_DENOM = np.float32(4.0 / 3.0 * np.pi * _MIN_RADIUS ** 3 + 1e-08)


def _dist_block(pts_ref, ctr_ref):
    p = pts_ref[0]
    c = ctr_ref[0]
    dx = c[:, 0:1] - p[0:1, :]
    dy = c[:, 1:2] - p[1:2, :]
    dz = c[:, 2:3] - p[2:3, :]
    return jnp.sqrt(dx * dx + dy * dy + dz * dz)


def _count_body(pts_ref, ctr_ref, cnt_ref):
    dist = _dist_block(pts_ref, ctr_ref)
    mask = (dist < _MIN_RADIUS).astype(jnp.float32)
    cnt_ref[0, 0] = jnp.sum(mask, axis=1, keepdims=True)


_SUP = np.float32(3e38)
_BIGI = np.float32(1e9)
_R = 3


def _select_body(pts_ref, ctr_ref, cnt_blk_ref, cnt_full_ref, out_ref, v_ref, v2_ref):
    n = pts_ref.shape[2]
    nc = n // 128
    dist = _dist_block(pts_ref, ctr_ref)
    density_full = cnt_full_ref[...] / _DENOM
    density_max = jnp.max(density_full) + np.float32(1e-8)
    density = cnt_blk_ref[0, 0] / _DENOM
    radii = _MIN_RADIUS + (_MAX_RADIUS - _MIN_RADIUS) * (1.0 - density / density_max)
    v_ref[...] = jnp.where(dist < radii, dist, jnp.float32(1e10))
    b_off = pl.program_id(0) * n

    lane_iota = lax.broadcasted_iota(jnp.int32, (_BM, 128), 1).astype(jnp.float32)
    v2_ref[...] = v_ref[...]
    lane_v = []
    lane_i = []
    for _ in range(_R):
        v2 = v2_ref[...]
        m = v2[:, 0:128]
        for a in range(1, nc):
            m = jnp.minimum(m, v2[:, a * 128 : (a + 1) * 128])
        ji = jnp.full((_BM, 128), _BIGI, jnp.float32)
        for a in range(nc):
            ch = v2[:, a * 128 : (a + 1) * 128]
            ji = jnp.minimum(
                ji, jnp.where(ch == m, lane_iota + np.float32(a * 128), _BIGI)
            )
        lane_v.append(m)
        lane_i.append(ji)
        for a in range(nc):
            ch = v2[:, a * 128 : (a + 1) * 128]
            v2_ref[:, a * 128 : (a + 1) * 128] = jnp.where(
                lane_iota + np.float32(a * 128) == ji, _SUP, ch
            )

    w, wi = lane_v[0], lane_i[0]
    lvl = jnp.zeros((_BM, 128), jnp.float32)
    t = None
    for k in range(_K):
        m = jnp.min(w, axis=1, keepdims=True)
        cand = jnp.where(w == m, wi, _BIGI)
        ji = jnp.min(cand, axis=1, keepdims=True)
        out_ref[0, 0, :, k : k + 1] = ji.astype(jnp.int32) + b_off
        t = m
        wn = jnp.where(lvl == 0.0, lane_v[1], jnp.where(lvl == 1.0, lane_v[2], _SUP))
        win = jnp.where(lvl == 0.0, lane_i[1], jnp.where(lvl == 1.0, lane_i[2], _BIGI))
        hit = (w == m) & (wi == ji)
        w = jnp.where(hit, wn, w)
        wi = jnp.where(hit, win, wi)
        lvl = lvl + hit.astype(jnp.float32)

    v2 = v2_ref[...]
    rem = jnp.zeros((_BM, 128), jnp.float32)
    for a in range(nc):
        ch = v2[:, a * 128 : (a + 1) * 128]
        rem = rem + (ch <= t).astype(jnp.float32)
    suspect = (lvl >= float(_R)) & (rem >= 1.0)
    any_suspect = jnp.max(suspect.astype(jnp.int32))

    @pl.when(any_suspect == 1)
    def _slow():
        iota = lax.broadcasted_iota(jnp.int32, (_BM, n), 1).astype(jnp.float32)
        for k in range(_K):
            v = v_ref[...]
            m = jnp.min(v, axis=1, keepdims=True)
            cand = jnp.where(v == m, iota, _BIGI)
            ji = jnp.min(cand, axis=1, keepdims=True)
            out_ref[0, 0, :, k : k + 1] = ji.astype(jnp.int32) + b_off
            v_ref[...] = jnp.where(iota == ji, _SUP, v)


def _sc_gather_body(per_w, ch, feat_ref, idx_ref, out_ref, idx_v, rows_v, sem):
    nc = lax.axis_size("c")
    wid = lax.axis_index("s") * nc + lax.axis_index("c")
    base = wid * per_w
    for i in range(per_w // ch):
        off = base + i * ch
        pltpu.sync_copy(idx_ref.at[pl.ds(off, ch)], idx_v)
        pltpu.async_copy(feat_ref.at[idx_v], rows_v, sem).wait()
        pltpu.sync_copy(rows_v, out_ref.at[pl.ds(off, ch)])


def kernel(points, features, center_indices):
    B, N, _ = points.shape
    M = center_indices.shape[1]
    C = features.shape[2]
    MB = M // _BM

    pointsT = points.transpose(0, 2, 1)
    centers = jnp.take_along_axis(
        points, jnp.broadcast_to(center_indices[:, :, None], (B, M, 3)), axis=1
    )

    counts = pl.pallas_call(
        _count_body,
        grid=(B, MB),
        in_specs=[
            pl.BlockSpec((1, 3, N), lambda b, mb: (b, 0, 0)),
            pl.BlockSpec((1, _BM, 3), lambda b, mb: (b, mb, 0)),
        ],
        out_specs=pl.BlockSpec((1, 1, _BM, 1), lambda b, mb: (b, mb, 0, 0)),
        out_shape=jax.ShapeDtypeStruct((B, MB, _BM, 1), jnp.float32),
    )(pointsT, centers)

    knn_idx = pl.pallas_call(
        _select_body,
        grid=(B, MB),
        in_specs=[
            pl.BlockSpec((1, 3, N), lambda b, mb: (b, 0, 0)),
            pl.BlockSpec((1, _BM, 3), lambda b, mb: (b, mb, 0)),
            pl.BlockSpec((1, 1, _BM, 1), lambda b, mb: (b, mb, 0, 0)),
            pl.BlockSpec((B, MB, _BM, 1), lambda b, mb: (0, 0, 0, 0)),
        ],
        out_specs=pl.BlockSpec((1, 1, _BM, _K), lambda b, mb: (b, mb, 0, 0)),
        out_shape=jax.ShapeDtypeStruct((B, MB, _BM, _K), jnp.int32),
        scratch_shapes=[
            pltpu.VMEM((_BM, N), jnp.float32),
            pltpu.VMEM((_BM, N), jnp.float32),
        ],
    )(pointsT, centers, counts, counts)

    tot = B * M * _K
    idx_flat = knn_idx.reshape(tot)
    feat_flat = features.reshape(B * N, C)

    info = plsc.get_sparse_core_info()
    nw = info.num_cores * info.num_subcores
    per_w = tot // nw
    ch = 128
    gather = pl.kernel(
        functools.partial(_sc_gather_body, per_w, ch),
        out_type=jax.ShapeDtypeStruct((tot, C), jnp.float32),
        mesh=plsc.VectorSubcoreMesh(core_axis_name="c", subcore_axis_name="s"),
        compiler_params=pltpu.CompilerParams(use_tc_tiling_on_sc=False),
        scratch_types=[
            pltpu.VMEM((ch,), jnp.int32),
            pltpu.VMEM((ch, C), jnp.float32),
            pltpu.SemaphoreType.DMA,
        ],
    )
    out_flat = gather(feat_flat, idx_flat)
    return out_flat.reshape(B, M, _K, C)
